# baseline (speedup 1.0000x reference)
import math

import jax
import jax.numpy as jnp
from jax.experimental import pallas as pl

_N_HASHES = 2
_NUM_REGIONS = 16
_BLOCK = 128
_H = 8
_D = 64
_NWPD = 8

_HI = jax.lax.Precision.HIGHEST
_F32 = jnp.float32


def _dot(a, b):
    return jax.lax.dot_general(a, b, (((1,), (0,)), ((), ())),
                               precision=_HI, preferred_element_type=_F32)


def _dot_t(a, b):
    return jax.lax.dot_general(a, b, (((1,), (1,)), ((), ())),
                               precision=_HI, preferred_element_type=_F32)


def _ln(x, s, b, eps=1e-5):
    m = jnp.mean(x, axis=-1, keepdims=True)
    xc = x - m
    v = jnp.mean(xc * xc, axis=-1, keepdims=True)
    return xc * jax.lax.rsqrt(v + eps) * s + b


def _full(a):
    return pl.BlockSpec(a.shape, lambda i, _n=a.ndim: (0,) * _n)


def _enc_body(x_ref, w1_ref, b1_ref, w2_ref, b2_ref, ns_ref, nb_ref,
              h_ref, xn_ref):
    x = x_ref[...]
    t = jnp.maximum(_dot(x, w1_ref[...]) + b1_ref[...], 0.0)
    h = _dot(t, w2_ref[...]) + b2_ref[...]
    h_ref[...] = h
    xn_ref[...] = _ln(h, ns_ref[...], nb_ref[...])


def _attn_body(xn_ref, ft_ref, bc_ref, br_ref, wq_ref, wk_ref, wv_ref,
               rw_ref, rb_ref, out_ref, den_ref):
    a = xn_ref[0]
    f = ft_ref[0]
    mask = bc_ref[0] == br_ref[0]
    q = _dot(a, wq_ref[...])
    k = _dot(a, wk_ref[...])
    v = _dot(a, wv_ref[...])
    pe = _dot(f, rw_ref[...]) + rb_ref[...]
    inv_scale = 1.0 / math.sqrt(2.0 * _D)
    dens = []
    for h in range(_H):
        sl = slice(h * _D, (h + 1) * _D)
        qh, kh, vh, ph = q[:, sl], k[:, sl], v[:, sl], pe[:, sl]
        s = (_dot_t(qh, kh) - _dot_t(ph, ph)) * inv_scale
        p = jnp.where(mask, jnp.exp(s), 0.0)
        out_ref[0, :, sl] = _dot(p, vh)
        dens.append(jnp.sum(p, axis=1, keepdims=True))
    den_ref[0] = jnp.concatenate(dens, axis=1)


def _post_body(h_ref, o_ref, de_ref, wo_ref, n2s_ref, n2b_ref,
               fw1_ref, fb1_ref, fw2_ref, fb2_ref, nxs_ref, nxb_ref,
               h_out_ref, xn_out_ref):
    o = o_ref[...]
    rec = 1.0 / (de_ref[...] + 1e-6)
    parts = []
    for h in range(_H):
        sl = slice(h * _D, (h + 1) * _D)
        parts.append(o[:, sl] * rec[:, h:h + 1])
    od = jnp.concatenate(parts, axis=1)
    hh = h_ref[...] + _dot(od, wo_ref[...])
    x2 = _ln(hh, n2s_ref[...], n2b_ref[...])
    ff = _dot(jnp.maximum(_dot(x2, fw1_ref[...]) + fb1_ref[...], 0.0),
              fw2_ref[...]) + fb2_ref[...]
    hh = hh + ff
    h_out_ref[...] = hh
    xn_out_ref[...] = _ln(hh, nxs_ref[...], nxb_ref[...])


def _final_body(h0_ref, h1_ref, h2_ref, h3_ref, w_ref,
                m0_ref, m1_ref, m2_ref, m3_ref, m4_ref,
                c0_ref, c1_ref, c2_ref, c3_ref, c4_ref,
                s0_ref, s1_ref, s2_ref, s3_ref,
                t0_ref, t1_ref, t2_ref, t3_ref,
                out_ref):
    wv = w_ref[...]
    enc = (_dot(h0_ref[...], wv[0:64]) + _dot(h1_ref[...], wv[64:128])
           + _dot(h2_ref[...], wv[128:192]) + _dot(h3_ref[...], wv[192:256]))
    z = enc
    ws = (m0_ref, m1_ref, m2_ref, m3_ref, m4_ref)
    bs = (c0_ref, c1_ref, c2_ref, c3_ref, c4_ref)
    ss = (s0_ref, s1_ref, s2_ref, s3_ref)
    ts = (t0_ref, t1_ref, t2_ref, t3_ref)
    for i in range(5):
        z = _dot(z, ws[i][...]) + bs[i][...]
        if i < 4:
            z = jnp.tanh(_ln(z, ss[i][...], ts[i][...]))
    out_ref[...] = enc + z


def _qpart(sorted_idx, qw, n):
    w = qw.reshape(_N_HASHES, _NUM_REGIONS)
    w = w / w.sum(-1, keepdims=True)
    boundaries = jnp.ceil(jnp.cumsum(w, -1) * n)
    ranks = jnp.arange(n)
    region_of_rank = (ranks[None, :, None] >= boundaries[:, None, :]).sum(-1)
    region_of_rank = region_of_rank.astype(jnp.int32)
    out = jnp.zeros((_N_HASHES, n), dtype=jnp.int32)
    out = out.at[:, sorted_idx].set(region_of_rank)
    return out


def _row(v):
    return v.reshape(1, -1)


def kernel(x, coords, params, regions, batch):
    raw = x.shape[0]
    pad = (-raw) % _BLOCK
    n = raw + pad
    if pad:
        xp = jnp.pad(x, ((0, pad), (0, 0)))
        cp = jnp.pad(coords, ((0, pad), (0, 0)), constant_values=jnp.inf)
    else:
        xp, cp = x, coords

    sorted_eta = jnp.argsort(cp[:, 0])
    sorted_phi = jnp.argsort(cp[:, 1])
    regions_h = jnp.transpose(regions, (1, 0, 2)).reshape(2, -1)
    reg_eta = _qpart(sorted_eta, regions_h[0], n)
    reg_phi = _qpart(sorted_phi, regions_h[1], n)
    if pad:
        cp = cp.at[raw:].set(0.0)
    buckets = reg_eta * _NUM_REGIONS + reg_phi
    if pad:
        buckets = jnp.where(jnp.arange(n)[None, :] < raw, buckets,
                            _NUM_REGIONS * _NUM_REGIONS + 7)

    r = jnp.arange(n)
    order0 = jnp.argsort(buckets[0] * (n + 1) + r)
    order1 = jnp.argsort(buckets[1] * (n + 1) + r)
    inv0 = jnp.argsort(order0)
    inv1 = jnp.argsort(order1)
    g01 = inv0[order1]
    g10 = inv1[order0]

    nb = n // _BLOCK
    gg = _N_HASHES * nb

    pe2 = cp[:, 1:3]
    feats = jnp.concatenate([pe2 ** (i + 1) for i in range(_NWPD)], axis=-1)
    xs = xp[order0]
    feats0 = feats[order0]
    fts3 = jnp.concatenate([feats0, feats0[g01]], 0).reshape(gg, _BLOCK, 16)
    bsort = jnp.concatenate([buckets[0][order0], buckets[1][order1]], 0)
    bc3 = bsort.reshape(gg, _BLOCK, 1)
    br3 = bsort.reshape(gg, 1, _BLOCK)

    p = params
    lp0 = p["layers"][0]
    h_s, xn_s = pl.pallas_call(
        _enc_body,
        grid=(nb,),
        in_specs=[pl.BlockSpec((_BLOCK, 64), lambda i: (i, 0))]
        + [_full(a) for a in (p["feat_w1"], _row(p["feat_b1"]),
                              p["feat_w2"], _row(p["feat_b2"]),
                              _row(lp0["norm1_s"]), _row(lp0["norm1_b"]))],
        out_specs=[pl.BlockSpec((_BLOCK, 64), lambda i: (i, 0))] * 2,
        out_shape=[jax.ShapeDtypeStruct((n, 64), _F32)] * 2,
    )(xs, p["feat_w1"], _row(p["feat_b1"]), p["feat_w2"], _row(p["feat_b2"]),
      _row(lp0["norm1_s"]), _row(lp0["norm1_b"]))

    hs_list = [h_s]
    n_layers = len(p["layers"])
    for li in range(n_layers):
        lp = p["layers"][li]
        xns3 = jnp.concatenate([xn_s, xn_s[g01]], 0).reshape(gg, _BLOCK, 64)
        out_s, den_s = pl.pallas_call(
            _attn_body,
            grid=(gg,),
            in_specs=[
                pl.BlockSpec((1, _BLOCK, 64), lambda i: (i, 0, 0)),
                pl.BlockSpec((1, _BLOCK, 16), lambda i: (i, 0, 0)),
                pl.BlockSpec((1, _BLOCK, 1), lambda i: (i, 0, 0)),
                pl.BlockSpec((1, 1, _BLOCK), lambda i: (i, 0, 0)),
                _full(lp["wq"]), _full(lp["wk"]), _full(lp["wv"]),
                _full(lp["rpe_w"]), _full(_row(lp["rpe_b"])),
            ],
            out_specs=[
                pl.BlockSpec((1, _BLOCK, _H * _D), lambda i: (i, 0, 0)),
                pl.BlockSpec((1, _BLOCK, _H), lambda i: (i, 0, 0)),
            ],
            out_shape=[
                jax.ShapeDtypeStruct((gg, _BLOCK, _H * _D), _F32),
                jax.ShapeDtypeStruct((gg, _BLOCK, _H), _F32),
            ],
        )(xns3, fts3, bc3, br3, lp["wq"], lp["wk"], lp["wv"],
          lp["rpe_w"], _row(lp["rpe_b"]))

        out_flat = out_s.reshape(_N_HASHES * n, _H * _D)
        den_flat = den_s.reshape(_N_HASHES * n, _H)
        o_acc = out_flat[:n] + out_flat[n:][g10]
        d_acc = den_flat[:n] + den_flat[n:][g10]

        if li + 1 < n_layers:
            nxs = p["layers"][li + 1]["norm1_s"]
            nxb = p["layers"][li + 1]["norm1_b"]
        else:
            nxs = jnp.ones((64,), _F32)
            nxb = jnp.zeros((64,), _F32)
        h_s, xn_s = pl.pallas_call(
            _post_body,
            grid=(nb,),
            in_specs=[
                pl.BlockSpec((_BLOCK, 64), lambda i: (i, 0)),
                pl.BlockSpec((_BLOCK, _H * _D), lambda i: (i, 0)),
                pl.BlockSpec((_BLOCK, _H), lambda i: (i, 0)),
            ] + [_full(a) for a in (
                lp["wo"], _row(lp["norm2_s"]), _row(lp["norm2_b"]),
                lp["ff_w1"], _row(lp["ff_b1"]), lp["ff_w2"], _row(lp["ff_b2"]),
                _row(nxs), _row(nxb))],
            out_specs=[pl.BlockSpec((_BLOCK, 64), lambda i: (i, 0))] * 2,
            out_shape=[jax.ShapeDtypeStruct((n, 64), _F32)] * 2,
        )(h_s, o_acc, d_acc, lp["wo"], _row(lp["norm2_s"]),
          _row(lp["norm2_b"]), lp["ff_w1"], _row(lp["ff_b1"]),
          lp["ff_w2"], _row(lp["ff_b2"]), _row(nxs), _row(nxb))
        hs_list.append(h_s)

    mp = p["mlp"]
    out_dim = p["W"].shape[1]
    outp_s = pl.pallas_call(
        _final_body,
        grid=(nb,),
        in_specs=[pl.BlockSpec((_BLOCK, 64), lambda i: (i, 0))] * 4
        + [_full(p["W"])]
        + [_full(w) for w in mp["w"]]
        + [_full(_row(b)) for b in mp["b"]]
        + [_full(_row(s)) for s in mp["ns"]]
        + [_full(_row(b)) for b in mp["nb"]],
        out_specs=pl.BlockSpec((_BLOCK, out_dim), lambda i: (i, 0)),
        out_shape=jax.ShapeDtypeStruct((n, out_dim), _F32),
    )(*hs_list, p["W"], *mp["w"], *[_row(b) for b in mp["b"]],
      *[_row(s) for s in mp["ns"]], *[_row(b) for b in mp["nb"]])

    return outp_s[inv0][:raw]


# speedup vs baseline: 1.9317x; 1.9317x over previous
import math

import jax
import jax.numpy as jnp
from jax.experimental import pallas as pl

_N_HASHES = 2
_NUM_REGIONS = 16
_BLOCK = 128
_H = 8
_D = 64
_NWPD = 8

_HI = jax.lax.Precision.DEFAULT
_F32 = jnp.float32


def _dot(a, b):
    return jax.lax.dot_general(a, b, (((1,), (0,)), ((), ())),
                               precision=_HI, preferred_element_type=_F32)


def _dot_t(a, b):
    return jax.lax.dot_general(a, b, (((1,), (1,)), ((), ())),
                               precision=_HI, preferred_element_type=_F32)


def _ln(x, s, b, eps=1e-5):
    m = jnp.mean(x, axis=-1, keepdims=True)
    xc = x - m
    v = jnp.mean(xc * xc, axis=-1, keepdims=True)
    return xc * jax.lax.rsqrt(v + eps) * s + b


def _full(a):
    return pl.BlockSpec(a.shape, lambda i, _n=a.ndim: (0,) * _n)


def _enc_body(x_ref, w1_ref, b1_ref, w2_ref, b2_ref, ns_ref, nb_ref,
              h_ref, xn_ref):
    x = x_ref[...]
    t = jnp.maximum(_dot(x, w1_ref[...]) + b1_ref[...], 0.0)
    h = _dot(t, w2_ref[...]) + b2_ref[...]
    h_ref[...] = h
    xn_ref[...] = _ln(h, ns_ref[...], nb_ref[...])


def _attn_body(xn_ref, ft_ref, bc_ref, br_ref, wq_ref, wk_ref, wv_ref,
               rw_ref, rb_ref, out_ref, den_ref):
    a = xn_ref[0]
    f = ft_ref[0]
    mask = bc_ref[0] == br_ref[0]
    q = _dot(a, wq_ref[...])
    k = _dot(a, wk_ref[...])
    v = _dot(a, wv_ref[...])
    pe = _dot(f, rw_ref[...]) + rb_ref[...]
    inv_scale = 1.0 / math.sqrt(2.0 * _D)
    dens = []
    for h in range(_H):
        sl = slice(h * _D, (h + 1) * _D)
        qh, kh, vh, ph = q[:, sl], k[:, sl], v[:, sl], pe[:, sl]
        s = (_dot_t(qh, kh) - _dot_t(ph, ph)) * inv_scale
        p = jnp.where(mask, jnp.exp(s), 0.0)
        out_ref[0, :, sl] = _dot(p, vh)
        dens.append(jnp.sum(p, axis=1, keepdims=True))
    den_ref[0] = jnp.concatenate(dens, axis=1)


def _post_body(h_ref, o_ref, de_ref, wo_ref, n2s_ref, n2b_ref,
               fw1_ref, fb1_ref, fw2_ref, fb2_ref, nxs_ref, nxb_ref,
               h_out_ref, xn_out_ref):
    o = o_ref[...]
    rec = 1.0 / (de_ref[...] + 1e-6)
    parts = []
    for h in range(_H):
        sl = slice(h * _D, (h + 1) * _D)
        parts.append(o[:, sl] * rec[:, h:h + 1])
    od = jnp.concatenate(parts, axis=1)
    hh = h_ref[...] + _dot(od, wo_ref[...])
    x2 = _ln(hh, n2s_ref[...], n2b_ref[...])
    ff = _dot(jnp.maximum(_dot(x2, fw1_ref[...]) + fb1_ref[...], 0.0),
              fw2_ref[...]) + fb2_ref[...]
    hh = hh + ff
    h_out_ref[...] = hh
    xn_out_ref[...] = _ln(hh, nxs_ref[...], nxb_ref[...])


def _final_body(h0_ref, h1_ref, h2_ref, h3_ref, w_ref,
                m0_ref, m1_ref, m2_ref, m3_ref, m4_ref,
                c0_ref, c1_ref, c2_ref, c3_ref, c4_ref,
                s0_ref, s1_ref, s2_ref, s3_ref,
                t0_ref, t1_ref, t2_ref, t3_ref,
                out_ref):
    wv = w_ref[...]
    enc = (_dot(h0_ref[...], wv[0:64]) + _dot(h1_ref[...], wv[64:128])
           + _dot(h2_ref[...], wv[128:192]) + _dot(h3_ref[...], wv[192:256]))
    z = enc
    ws = (m0_ref, m1_ref, m2_ref, m3_ref, m4_ref)
    bs = (c0_ref, c1_ref, c2_ref, c3_ref, c4_ref)
    ss = (s0_ref, s1_ref, s2_ref, s3_ref)
    ts = (t0_ref, t1_ref, t2_ref, t3_ref)
    for i in range(5):
        z = _dot(z, ws[i][...]) + bs[i][...]
        if i < 4:
            z = jnp.tanh(_ln(z, ss[i][...], ts[i][...]))
    out_ref[...] = enc + z


def _qpart(sorted_idx, qw, n):
    w = qw.reshape(_N_HASHES, _NUM_REGIONS)
    w = w / w.sum(-1, keepdims=True)
    boundaries = jnp.ceil(jnp.cumsum(w, -1) * n)
    ranks = jnp.arange(n)
    region_of_rank = (ranks[None, :, None] >= boundaries[:, None, :]).sum(-1)
    region_of_rank = region_of_rank.astype(jnp.int32)
    out = jnp.zeros((_N_HASHES, n), dtype=jnp.int32)
    out = out.at[:, sorted_idx].set(region_of_rank)
    return out


def _row(v):
    return v.reshape(1, -1)


def kernel(x, coords, params, regions, batch):
    raw = x.shape[0]
    pad = (-raw) % _BLOCK
    n = raw + pad
    if pad:
        xp = jnp.pad(x, ((0, pad), (0, 0)))
        cp = jnp.pad(coords, ((0, pad), (0, 0)), constant_values=jnp.inf)
    else:
        xp, cp = x, coords

    sorted_eta = jnp.argsort(cp[:, 0])
    sorted_phi = jnp.argsort(cp[:, 1])
    regions_h = jnp.transpose(regions, (1, 0, 2)).reshape(2, -1)
    reg_eta = _qpart(sorted_eta, regions_h[0], n)
    reg_phi = _qpart(sorted_phi, regions_h[1], n)
    if pad:
        cp = cp.at[raw:].set(0.0)
    buckets = reg_eta * _NUM_REGIONS + reg_phi
    if pad:
        buckets = jnp.where(jnp.arange(n)[None, :] < raw, buckets,
                            _NUM_REGIONS * _NUM_REGIONS + 7)

    r = jnp.arange(n)
    order0 = jnp.argsort(buckets[0] * (n + 1) + r)
    order1 = jnp.argsort(buckets[1] * (n + 1) + r)
    inv0 = jnp.argsort(order0)
    inv1 = jnp.argsort(order1)
    g01 = inv0[order1]
    g10 = inv1[order0]

    nb = n // _BLOCK
    gg = _N_HASHES * nb

    pe2 = cp[:, 1:3]
    feats = jnp.concatenate([pe2 ** (i + 1) for i in range(_NWPD)], axis=-1)
    xs = xp[order0]
    feats0 = feats[order0]
    fts3 = jnp.concatenate([feats0, feats0[g01]], 0).reshape(gg, _BLOCK, 16)
    bsort = jnp.concatenate([buckets[0][order0], buckets[1][order1]], 0)
    bc3 = bsort.reshape(gg, _BLOCK, 1)
    br3 = bsort.reshape(gg, 1, _BLOCK)

    p = params
    lp0 = p["layers"][0]
    h_s, xn_s = pl.pallas_call(
        _enc_body,
        grid=(nb,),
        in_specs=[pl.BlockSpec((_BLOCK, 64), lambda i: (i, 0))]
        + [_full(a) for a in (p["feat_w1"], _row(p["feat_b1"]),
                              p["feat_w2"], _row(p["feat_b2"]),
                              _row(lp0["norm1_s"]), _row(lp0["norm1_b"]))],
        out_specs=[pl.BlockSpec((_BLOCK, 64), lambda i: (i, 0))] * 2,
        out_shape=[jax.ShapeDtypeStruct((n, 64), _F32)] * 2,
    )(xs, p["feat_w1"], _row(p["feat_b1"]), p["feat_w2"], _row(p["feat_b2"]),
      _row(lp0["norm1_s"]), _row(lp0["norm1_b"]))

    hs_list = [h_s]
    n_layers = len(p["layers"])
    for li in range(n_layers):
        lp = p["layers"][li]
        xns3 = jnp.concatenate([xn_s, xn_s[g01]], 0).reshape(gg, _BLOCK, 64)
        out_s, den_s = pl.pallas_call(
            _attn_body,
            grid=(gg,),
            in_specs=[
                pl.BlockSpec((1, _BLOCK, 64), lambda i: (i, 0, 0)),
                pl.BlockSpec((1, _BLOCK, 16), lambda i: (i, 0, 0)),
                pl.BlockSpec((1, _BLOCK, 1), lambda i: (i, 0, 0)),
                pl.BlockSpec((1, 1, _BLOCK), lambda i: (i, 0, 0)),
                _full(lp["wq"]), _full(lp["wk"]), _full(lp["wv"]),
                _full(lp["rpe_w"]), _full(_row(lp["rpe_b"])),
            ],
            out_specs=[
                pl.BlockSpec((1, _BLOCK, _H * _D), lambda i: (i, 0, 0)),
                pl.BlockSpec((1, _BLOCK, _H), lambda i: (i, 0, 0)),
            ],
            out_shape=[
                jax.ShapeDtypeStruct((gg, _BLOCK, _H * _D), _F32),
                jax.ShapeDtypeStruct((gg, _BLOCK, _H), _F32),
            ],
        )(xns3, fts3, bc3, br3, lp["wq"], lp["wk"], lp["wv"],
          lp["rpe_w"], _row(lp["rpe_b"]))

        out_flat = out_s.reshape(_N_HASHES * n, _H * _D)
        den_flat = den_s.reshape(_N_HASHES * n, _H)
        o_acc = out_flat[:n] + out_flat[n:][g10]
        d_acc = den_flat[:n] + den_flat[n:][g10]

        if li + 1 < n_layers:
            nxs = p["layers"][li + 1]["norm1_s"]
            nxb = p["layers"][li + 1]["norm1_b"]
        else:
            nxs = jnp.ones((64,), _F32)
            nxb = jnp.zeros((64,), _F32)
        h_s, xn_s = pl.pallas_call(
            _post_body,
            grid=(nb,),
            in_specs=[
                pl.BlockSpec((_BLOCK, 64), lambda i: (i, 0)),
                pl.BlockSpec((_BLOCK, _H * _D), lambda i: (i, 0)),
                pl.BlockSpec((_BLOCK, _H), lambda i: (i, 0)),
            ] + [_full(a) for a in (
                lp["wo"], _row(lp["norm2_s"]), _row(lp["norm2_b"]),
                lp["ff_w1"], _row(lp["ff_b1"]), lp["ff_w2"], _row(lp["ff_b2"]),
                _row(nxs), _row(nxb))],
            out_specs=[pl.BlockSpec((_BLOCK, 64), lambda i: (i, 0))] * 2,
            out_shape=[jax.ShapeDtypeStruct((n, 64), _F32)] * 2,
        )(h_s, o_acc, d_acc, lp["wo"], _row(lp["norm2_s"]),
          _row(lp["norm2_b"]), lp["ff_w1"], _row(lp["ff_b1"]),
          lp["ff_w2"], _row(lp["ff_b2"]), _row(nxs), _row(nxb))
        hs_list.append(h_s)

    mp = p["mlp"]
    out_dim = p["W"].shape[1]
    outp_s = pl.pallas_call(
        _final_body,
        grid=(nb,),
        in_specs=[pl.BlockSpec((_BLOCK, 64), lambda i: (i, 0))] * 4
        + [_full(p["W"])]
        + [_full(w) for w in mp["w"]]
        + [_full(_row(b)) for b in mp["b"]]
        + [_full(_row(s)) for s in mp["ns"]]
        + [_full(_row(b)) for b in mp["nb"]],
        out_specs=pl.BlockSpec((_BLOCK, out_dim), lambda i: (i, 0)),
        out_shape=jax.ShapeDtypeStruct((n, out_dim), _F32),
    )(*hs_list, p["W"], *mp["w"], *[_row(b) for b in mp["b"]],
      *[_row(s) for s in mp["ns"]], *[_row(b) for b in mp["nb"]])

    return outp_s[inv0][:raw]
